# baseline (speedup 1.0000x reference)
import functools

import jax
import jax.numpy as jnp
from jax import lax
from jax.experimental import pallas as pl
from jax.experimental.pallas import tpu as pltpu
from jax.experimental.pallas import tpu_sc as plsc

VOCAB = 100000
EMB = 128
B = 4096
L = 50
TOT = B * L
NC = 2
NS = 16
NW = NC * NS
PER_W = TOT // NW
C = 128
NCH = PER_W // C
NB = 6
NG = 8
NSLOT = 2

_mesh = plsc.VectorSubcoreMesh(core_axis_name="c", subcore_axis_name="s")


@functools.partial(
    pl.kernel,
    out_type=jax.ShapeDtypeStruct((TOT, EMB), jnp.float32),
    mesh=_mesh,
    scratch_types=[
        pltpu.VMEM((NCH, C), jnp.int32),
        [pltpu.VMEM((C, EMB), jnp.float32) for _ in range(NB)],
        pltpu.VMEM_SHARED((NS, NSLOT, C // 2, EMB), jnp.float32),
        [pltpu.SemaphoreType.DMA for _ in range(NB)],
        [pltpu.SemaphoreType.DMA for _ in range(NSLOT)],
        [pltpu.SemaphoreType.DMA for _ in range(NSLOT)],
    ],
)
def _gather_kernel(table_hbm, idx_hbm, out_hbm, idx_v, bufs, spm,
                   gsems, csems, osems):
    cid = lax.axis_index("c")
    sid = lax.axis_index("s")
    wid = sid * NC + cid
    wbase = wid * PER_W
    pltpu.sync_copy(idx_hbm.at[wid], idx_v)

    H = C // 2

    def stage_and_write(g, b, gi_guard):
        pltpu.make_async_copy(
            table_hbm.at[idx_v.at[g]], bufs[b], gsems[b]
        ).wait()
        for h in range(2):
            s = h
            if gi_guard is None:
                pltpu.make_async_copy(
                    spm.at[sid, s], out_hbm.at[pl.ds(wbase, H)], osems[s]
                ).wait()
            else:
                @pl.when(gi_guard)
                def _():
                    pltpu.make_async_copy(
                        spm.at[sid, s], out_hbm.at[pl.ds(wbase, H)], osems[s]
                    ).wait()
            pltpu.async_copy(bufs[b].at[pl.ds(h * H, H)], spm.at[sid, s],
                             csems[s])
            pltpu.make_async_copy(bufs[b].at[pl.ds(h * H, H)],
                                  spm.at[sid, s], csems[s]).wait()
            pltpu.async_copy(spm.at[sid, s],
                             out_hbm.at[pl.ds(wbase + g * C + h * H, H)],
                             osems[s])

    def group(gi, carry):
        for b in range(NB):
            g = gi * NB + b
            pltpu.async_copy(table_hbm.at[idx_v.at[g]], bufs[b], gsems[b])
        for b in range(NB):
            g = gi * NB + b
            stage_and_write(g, b, (gi > 0) if b == 0 else None)
        return carry

    lax.fori_loop(0, NG, group, 0)
    for t in range(2):
        g = NG * NB + t
        pltpu.async_copy(table_hbm.at[idx_v.at[g]], bufs[t], gsems[t])
    for t in range(2):
        g = NG * NB + t
        stage_and_write(g, t, None)
    for s in range(NSLOT):
        pltpu.make_async_copy(
            spm.at[sid, s], out_hbm.at[pl.ds(wbase, H)], osems[s]
        ).wait()


def kernel(input_ids, table):
    idx = input_ids.astype(jnp.int32).T.reshape(NW, NCH, C)
    out = _gather_kernel(table, idx)
    return out.reshape(L, B, EMB).transpose(1, 0, 2)


# speedup vs baseline: 1.1082x; 1.1082x over previous
import functools

import jax
import jax.numpy as jnp
from jax import lax
from jax.experimental import pallas as pl
from jax.experimental.pallas import tpu as pltpu
from jax.experimental.pallas import tpu_sc as plsc

VOCAB = 100000
EMB = 128
B = 4096
L = 50
TOT = B * L
NC = 2
NS = 16
NW = NC * NS
PER_W = TOT // NW
C = 128
NCH = PER_W // C
NP = NCH // 2
NBB = 3
NG = 8

_mesh = plsc.VectorSubcoreMesh(core_axis_name="c", subcore_axis_name="s")


@functools.partial(
    pl.kernel,
    out_type=jax.ShapeDtypeStruct((TOT, EMB), jnp.float32),
    mesh=_mesh,
    scratch_types=[
        pltpu.VMEM((NCH, C), jnp.int32),
        [pltpu.VMEM((2 * C, EMB), jnp.float32) for _ in range(NBB)],
        [pltpu.SemaphoreType.DMA for _ in range(NBB)],
        [pltpu.SemaphoreType.DMA for _ in range(NBB)],
    ],
)
def _gather_kernel(table_hbm, idx_hbm, out_hbm, idx_v, bufs, gsems, osems):
    wid = lax.axis_index("s") * NC + lax.axis_index("c")
    wbase = wid * PER_W
    pltpu.sync_copy(idx_hbm.at[wid], idx_v)

    def issue_pair(p, b):
        pltpu.async_copy(table_hbm.at[idx_v.at[2 * p]],
                         bufs[b].at[pl.ds(0, C)], gsems[b])
        pltpu.async_copy(table_hbm.at[idx_v.at[2 * p + 1]],
                         bufs[b].at[pl.ds(C, C)], gsems[b])

    def wait_pair(p, b):
        pltpu.make_async_copy(table_hbm.at[idx_v.at[2 * p]],
                              bufs[b].at[pl.ds(0, C)], gsems[b]).wait()
        pltpu.make_async_copy(table_hbm.at[idx_v.at[2 * p + 1]],
                              bufs[b].at[pl.ds(C, C)], gsems[b]).wait()

    def group(gi, carry):
        for b in range(NBB):
            p = gi * NBB + b

            @pl.when(gi > 0)
            def _():
                pltpu.make_async_copy(
                    bufs[b], out_hbm.at[pl.ds(wbase, 2 * C)], osems[b]
                ).wait()

            issue_pair(p, b)
        for b in range(NBB):
            p = gi * NBB + b
            wait_pair(p, b)
            pltpu.async_copy(
                bufs[b], out_hbm.at[pl.ds(wbase + 2 * p * C, 2 * C)],
                osems[b])
        return carry

    lax.fori_loop(0, NG, group, 0)
    pltpu.make_async_copy(
        bufs[0], out_hbm.at[pl.ds(wbase, 2 * C)], osems[0]
    ).wait()
    issue_pair(NP - 1, 0)
    wait_pair(NP - 1, 0)
    pltpu.async_copy(
        bufs[0], out_hbm.at[pl.ds(wbase + 2 * (NP - 1) * C, 2 * C)], osems[0])
    pltpu.make_async_copy(
        bufs[0], out_hbm.at[pl.ds(wbase, 2 * C)], osems[0]
    ).wait()
    for b in range(1, NBB):
        pltpu.make_async_copy(
            bufs[b], out_hbm.at[pl.ds(wbase, 2 * C)], osems[b]
        ).wait()


def kernel(input_ids, table):
    idx = input_ids.astype(jnp.int32).T.reshape(NW, NCH, C)
    out = _gather_kernel(table, idx)
    return out.reshape(L, B, EMB).transpose(1, 0, 2)


# speedup vs baseline: 1.1491x; 1.0369x over previous
import functools

import jax
import jax.numpy as jnp
from jax import lax
from jax.experimental import pallas as pl
from jax.experimental.pallas import tpu as pltpu
from jax.experimental.pallas import tpu_sc as plsc

VOCAB = 100000
EMB = 128
B = 4096
L = 50
TOT = B * L
NC = 2
NS = 16
NW = NC * NS
PER_W = TOT // NW
C = 128
NCH = PER_W // C
NB = 5
NG = NCH // NB

_mesh = plsc.VectorSubcoreMesh(core_axis_name="c", subcore_axis_name="s")


@functools.partial(
    pl.kernel,
    out_type=jax.ShapeDtypeStruct((TOT, EMB), jnp.float32),
    mesh=_mesh,
    scratch_types=[
        pltpu.VMEM((NCH, C), jnp.int32),
        [pltpu.VMEM((C, EMB), jnp.float32) for _ in range(NB)],
        [pltpu.SemaphoreType.DMA for _ in range(NB)],
        [pltpu.SemaphoreType.DMA for _ in range(NB)],
    ],
)
def _gather_kernel(table_hbm, idx_hbm, out_hbm, idx_v, bufs, gsems, osems):
    wid = lax.axis_index("s") * NC + lax.axis_index("c")
    wbase = wid * PER_W
    pltpu.sync_copy(idx_hbm.at[wid], idx_v)

    def group(gi, carry):
        for b in range(NB):
            g = gi * NB + b

            @pl.when(gi > 0)
            def _():
                pltpu.make_async_copy(
                    bufs[b], out_hbm.at[pl.ds(wbase, C)], osems[b]
                ).wait()

            pltpu.async_copy(table_hbm.at[idx_v.at[g]], bufs[b], gsems[b])
        for b in range(NB):
            g = gi * NB + b
            pltpu.make_async_copy(
                table_hbm.at[idx_v.at[g]], bufs[b], gsems[b]
            ).wait()
            pltpu.async_copy(bufs[b], out_hbm.at[pl.ds(wbase + g * C, C)],
                             osems[b])
        return carry

    lax.fori_loop(0, NG, group, 0)
    for b in range(NB):
        pltpu.make_async_copy(
            bufs[b], out_hbm.at[pl.ds(wbase, C)], osems[b]
        ).wait()


def kernel(input_ids, table):
    idx = input_ids.astype(jnp.int32).T.reshape(NW, NCH, C)
    out = _gather_kernel(table, idx)
    return out.reshape(L, B, EMB).transpose(1, 0, 2)
